# Initial kernel scaffold; baseline (speedup 1.0000x reference)
import functools

import jax
import jax.numpy as jnp
from jax import lax
from jax.experimental import pallas as pl
from jax.experimental.pallas import tpu as pltpu
from jax.experimental.pallas import tpu_sc as plsc

B, T, H, W, D = 8, 4, 128, 128, 32
NW = 32
PER_W = (B * T * H * W) // NW
C = 1024
NCHUNK = PER_W // C
GATHER_BLK = 128


def _sc_body(x_hbm, idx0_hbm, idx1_hbm, w_hbm, b_hbm, t0_hbm, t1_hbm,
             out_hbm, idx0_v, idx1_v, x_v, rows0_v, rows1_v, outT_v,
             w_v, b_v, sem):
    wid = lax.axis_index("s") * 2 + lax.axis_index("c")
    pltpu.sync_copy(w_hbm, w_v)
    pltpu.sync_copy(b_hbm, b_v)

    def chunk(c, carry):
        base = c * C
        pltpu.sync_copy(idx0_hbm.at[wid, pl.ds(base, C)], idx0_v)
        pltpu.sync_copy(idx1_hbm.at[wid, pl.ds(base, C)], idx1_v)
        pltpu.sync_copy(x_hbm.at[wid, pl.ds(base, C)], x_v)
        cps = []
        for j in range(C // GATHER_BLK):
            s = pl.ds(j * GATHER_BLK, GATHER_BLK)
            cps.append(pltpu.async_copy(
                t0_hbm.at[idx0_v.at[s]], rows0_v.at[s, :], sem))
            cps.append(pltpu.async_copy(
                t1_hbm.at[idx1_v.at[s]], rows1_v.at[s, :], sem))
        for cp in cps:
            cp.wait()

        def grp(g, gcarry):
            i0 = g * 16
            rowi = i0 + lax.iota(jnp.int32, 16)
            xv = x_v[pl.ds(i0, 16)]
            for d in range(D):
                ci = jnp.full((16,), d, jnp.int32)
                r0 = plsc.load_gather(rows0_v, [rowi, ci])
                r1 = plsc.load_gather(rows1_v, [rowi, ci])
                outT_v[d, pl.ds(i0, 16)] = r0 + r1 + xv * w_v[d] + b_v[d]
            return gcarry

        lax.fori_loop(0, C // 16, grp, 0)
        pltpu.sync_copy(outT_v, out_hbm.at[wid, :, pl.ds(base, C)])
        return carry

    lax.fori_loop(0, NCHUNK, chunk, 0)


_sc_embed = functools.partial(
    pl.kernel,
    out_type=jax.ShapeDtypeStruct((NW, D, PER_W), jnp.float32),
    mesh=plsc.VectorSubcoreMesh(core_axis_name="c", subcore_axis_name="s"),
    scratch_types=[
        pltpu.VMEM((C,), jnp.int32),
        pltpu.VMEM((C,), jnp.int32),
        pltpu.VMEM((C,), jnp.float32),
        pltpu.VMEM((C, D), jnp.float32),
        pltpu.VMEM((C, D), jnp.float32),
        pltpu.VMEM((D, C), jnp.float32),
        pltpu.VMEM((D,), jnp.float32),
        pltpu.VMEM((D,), jnp.float32),
        pltpu.SemaphoreType.DMA,
    ],
)(_sc_body)


@jax.jit
def kernel(x_cont, idx_cat0, idx_cat1, cont_weight, cont_bias,
           cat0_table, cat1_table):
    x_f = x_cont.reshape(NW, PER_W)
    idx0_f = idx_cat0.reshape(NW, PER_W).astype(jnp.int32)
    idx1_f = idx_cat1.reshape(NW, PER_W).astype(jnp.int32)
    w_f = cont_weight.reshape(D)
    b_f = cont_bias.reshape(D)
    out = _sc_embed(x_f, idx0_f, idx1_f, w_f, b_f, cat0_table, cat1_table)
    return out.reshape(B, T, D, H, W)


# speedup vs baseline: 4.9516x; 4.9516x over previous
import functools

import jax
import jax.numpy as jnp
from jax import lax
from jax.experimental import pallas as pl
from jax.experimental.pallas import tpu as pltpu
from jax.experimental.pallas import tpu_sc as plsc

B, T, H, W, D = 8, 4, 128, 128, 32
N = B * T * H * W
NW = 32
PER_W = N // NW
C = 1024
NCHUNK = PER_W // C
GATHER_BLK = 128
UNROLL = 8


def _sc_body(idx0_hbm, idx1_hbm, t0_hbm, t1_hbm, g_hbm,
             idx0_v, idx1_v, rows0_v, rows1_v, sem):
    wid = lax.axis_index("s") * 2 + lax.axis_index("c")
    wbase = wid * PER_W

    def chunk(c, carry):
        base = c * C
        pltpu.sync_copy(idx0_hbm.at[pl.ds(wbase + base, C)], idx0_v)
        pltpu.sync_copy(idx1_hbm.at[pl.ds(wbase + base, C)], idx1_v)
        cps = []
        for j in range(C // GATHER_BLK):
            s = pl.ds(j * GATHER_BLK, GATHER_BLK)
            cps.append(pltpu.async_copy(
                t0_hbm.at[idx0_v.at[s]], rows0_v.at[s, :], sem))
            cps.append(pltpu.async_copy(
                t1_hbm.at[idx1_v.at[s]], rows1_v.at[s, :], sem))
        for cp in cps:
            cp.wait()

        def add_grp(g, gcarry):
            i0 = g * UNROLL
            for u in range(UNROLL):
                i = i0 + u
                for h in range(0, D, 16):
                    sl = pl.ds(h, 16)
                    rows0_v[i, sl] = rows0_v[i, sl] + rows1_v[i, sl]
            return gcarry

        lax.fori_loop(0, C // UNROLL, add_grp, 0)
        pltpu.sync_copy(rows0_v, g_hbm.at[pl.ds(wbase + base, C), :])
        return carry

    lax.fori_loop(0, NCHUNK, chunk, 0)


_sc_gather_sum = functools.partial(
    pl.kernel,
    out_type=jax.ShapeDtypeStruct((N, D), jnp.float32),
    mesh=plsc.VectorSubcoreMesh(core_axis_name="c", subcore_axis_name="s"),
    compiler_params=pltpu.CompilerParams(use_tc_tiling_on_sc=False),
    scratch_types=[
        pltpu.VMEM((C,), jnp.int32),
        pltpu.VMEM((C,), jnp.int32),
        pltpu.VMEM((C, D), jnp.float32),
        pltpu.VMEM((C, D), jnp.float32),
        pltpu.SemaphoreType.DMA,
    ],
)(_sc_body)


def _tc_body(g_ref, x_ref, w_ref, b_ref, out_ref):
    g = g_ref[...]
    eye = (lax.broadcasted_iota(jnp.int32, (D, D), 0)
           == lax.broadcasted_iota(jnp.int32, (D, D), 1)).astype(jnp.float32)
    gt = lax.dot_general(eye, g, (((1,), (1,)), ((), ())),
                         preferred_element_type=jnp.float32,
                         precision=lax.Precision.HIGHEST)
    xv = x_ref[0]
    wv = w_ref[0].reshape(D, 1)
    bv = b_ref[0].reshape(D, 1)
    out_ref[0] = gt + wv * xv + bv


def _tc_fma_transpose(g, x, w, b):
    return pl.pallas_call(
        _tc_body,
        grid=(NW,),
        in_specs=[
            pl.BlockSpec((PER_W, D), lambda i: (i, 0)),
            pl.BlockSpec((1, 1, PER_W), lambda i: (i, 0, 0)),
            pl.BlockSpec((1, D), lambda i: (0, 0)),
            pl.BlockSpec((1, D), lambda i: (0, 0)),
        ],
        out_specs=pl.BlockSpec((1, D, PER_W), lambda i: (i, 0, 0)),
        out_shape=jax.ShapeDtypeStruct((NW, D, PER_W), jnp.float32),
    )(g.reshape(N, D), x.reshape(NW, 1, PER_W), w, b)


@jax.jit
def kernel(x_cont, idx_cat0, idx_cat1, cont_weight, cont_bias,
           cat0_table, cat1_table):
    idx0_f = idx_cat0.reshape(N).astype(jnp.int32)
    idx1_f = idx_cat1.reshape(N).astype(jnp.int32)
    g = _sc_gather_sum(idx0_f, idx1_f, cat0_table, cat1_table)
    out = _tc_fma_transpose(g, x_cont.reshape(NW, PER_W),
                            cont_weight, cont_bias)
    return out.reshape(B, T, D, H, W)


# speedup vs baseline: 5.7553x; 1.1623x over previous
import functools

import jax
import jax.numpy as jnp
from jax import lax
from jax.experimental import pallas as pl
from jax.experimental.pallas import tpu as pltpu
from jax.experimental.pallas import tpu_sc as plsc

B, T, H, W, D = 8, 4, 128, 128, 32
N = B * T * H * W
NW = 32
PER_W = N // NW
C = 1024
NCHUNK = PER_W // C
GATHER_BLK = 128
UNROLL = 8


def _sc_body(idx0_hbm, idx1_hbm, t0_hbm, t1_hbm, g_hbm,
             idx0_v, idx1_v, rows0_v, rows1_v, sem):
    wid = lax.axis_index("s") * 2 + lax.axis_index("c")
    wbase = wid * PER_W

    def chunk(c, carry):
        base = c * C
        pltpu.sync_copy(idx0_hbm.at[pl.ds(wbase + base, C)], idx0_v)
        pltpu.sync_copy(idx1_hbm.at[pl.ds(wbase + base, C)], idx1_v)
        cps = []
        for j in range(C // GATHER_BLK):
            s = pl.ds(j * GATHER_BLK, GATHER_BLK)
            cps.append(pltpu.async_copy(
                t0_hbm.at[idx0_v.at[s]], rows0_v.at[s, :], sem))
            cps.append(pltpu.async_copy(
                t1_hbm.at[idx1_v.at[s]], rows1_v.at[s, :], sem))
        for cp in cps:
            cp.wait()

        def add_grp(g, gcarry):
            i0 = g * UNROLL
            for u in range(UNROLL):
                i = i0 + u
                for h in range(0, D, 16):
                    sl = pl.ds(h, 16)
                    rows0_v[i, sl] = rows0_v[i, sl] + rows1_v[i, sl]
            return gcarry

        lax.fori_loop(0, C // UNROLL, add_grp, 0)
        pltpu.sync_copy(rows0_v, g_hbm.at[pl.ds(wbase + base, C), :])
        return carry

    lax.fori_loop(0, NCHUNK, chunk, 0)


_sc_gather_sum = functools.partial(
    pl.kernel,
    out_type=jax.ShapeDtypeStruct((N, D), jnp.float32),
    mesh=plsc.VectorSubcoreMesh(core_axis_name="c", subcore_axis_name="s"),
    compiler_params=pltpu.CompilerParams(use_tc_tiling_on_sc=False),
    scratch_types=[
        pltpu.VMEM((C,), jnp.int32),
        pltpu.VMEM((C,), jnp.int32),
        pltpu.VMEM((C, D), jnp.float32),
        pltpu.VMEM((C, D), jnp.float32),
        pltpu.SemaphoreType.DMA,
    ],
)(_sc_body)


def _tc_body(g_ref, x_ref, w_ref, b_ref, out_ref):
    g = g_ref[...]
    gt = lax.transpose(g, (1, 0))
    xv = x_ref[0]
    wv = w_ref[0].reshape(D, 1)
    bv = b_ref[0].reshape(D, 1)
    out_ref[0] = gt + wv * xv + bv


CB = 4096
TC_SPLIT = PER_W // CB


def _tc_fma_transpose(g, x, w, b):
    return pl.pallas_call(
        _tc_body,
        grid=(NW, TC_SPLIT),
        in_specs=[
            pl.BlockSpec((CB, D), lambda i, j: (i * TC_SPLIT + j, 0)),
            pl.BlockSpec((1, 1, CB), lambda i, j: (i, 0, j)),
            pl.BlockSpec((1, D), lambda i, j: (0, 0)),
            pl.BlockSpec((1, D), lambda i, j: (0, 0)),
        ],
        out_specs=pl.BlockSpec((1, D, CB), lambda i, j: (i, 0, j)),
        out_shape=jax.ShapeDtypeStruct((NW, D, PER_W), jnp.float32),
    )(g.reshape(N, D), x.reshape(NW, 1, PER_W), w, b)


@jax.jit
def kernel(x_cont, idx_cat0, idx_cat1, cont_weight, cont_bias,
           cat0_table, cat1_table):
    idx0_f = idx_cat0.reshape(N).astype(jnp.int32)
    idx1_f = idx_cat1.reshape(N).astype(jnp.int32)
    g = _sc_gather_sum(idx0_f, idx1_f, cat0_table, cat1_table)
    out = _tc_fma_transpose(g, x_cont.reshape(NW, PER_W),
                            cont_weight, cont_bias)
    return out.reshape(B, T, D, H, W)
